# baseline (speedup 1.0000x reference)
import jax
import jax.numpy as jnp
from jax import lax
from jax.experimental import pallas as pl
from jax.experimental.pallas import tpu as pltpu
from jax.experimental.pallas import tpu_sc as plsc

N = 10000
E = 320000
F_IN = 128
HID = 64
C_OUT = 2

NCORES = 2
NSUB = 16
NW = NCORES * NSUB
EPT = E // NW
CHUNK = 128
EROWS = E // CHUNK
NCHB = EROWS // NW
EXTRA = EROWS % NW
OSL = N // NSUB
TAB2 = 2 * N

_MESH = plsc.VectorSubcoreMesh(core_axis_name="c", subcore_axis_name="s")
_SC_PARAMS = pltpu.CompilerParams(
    needs_layout_passes=False, use_tc_tiling_on_sc=False)


HR = N // 16
HCH = 125


def _hist_body(dst_hbm, iota_hbm, out_hbm, idx_v, iota_v, acc_v, acc_sh, sem):
    c = lax.axis_index("c")
    s = lax.axis_index("s")
    t = c * NSUB + s
    pltpu.async_copy(dst_hbm.at[pl.ds(t * EPT, EPT)], idx_v, sem).wait()
    pltpu.async_copy(iota_hbm, iota_v, sem).wait()
    zero16 = jnp.zeros((16,), jnp.float32)
    one16 = jnp.ones((16,), jnp.float32)

    def zbody(i, carry):
        acc_v[i, :] = zero16
        return carry

    lax.fori_loop(0, HR, zbody, 0)

    @pl.when(s == 0)
    def _():
        pltpu.sync_copy(acc_v, acc_sh)

    col16 = jnp.full((16,), 15, jnp.int32)

    def body(i, carry):
        d = idx_v[pl.ds(i * 16, 16)]
        plsc.addupdate_scatter(acc_v, [lax.shift_right_logical(d, 4),
                                       lax.bitwise_and(d, col16)], one16)
        return carry

    lax.fori_loop(0, EPT // 16, body, 0)
    plsc.subcore_barrier()
    for k in range(HR // HCH):
        pltpu.sync_copy(acc_v.at[pl.ds(k * HCH, HCH), :],
                        acc_sh.at[iota_v.at[k]], add=True)
    plsc.subcore_barrier()

    @pl.when(s == 0)
    def _():
        pltpu.sync_copy(acc_sh, out_hbm.at[c])


_hist = pl.kernel(
    _hist_body,
    mesh=_MESH,
    compiler_params=_SC_PARAMS,
    out_type=jax.ShapeDtypeStruct((NCORES, HR, 16), jnp.float32),
    scratch_types=[
        pltpu.VMEM((EPT,), jnp.int32),
        pltpu.VMEM((HR // HCH, HCH), jnp.int32),
        pltpu.VMEM((HR, 16), jnp.float32),
        pltpu.VMEM_SHARED((HR, 16), jnp.float32),
        pltpu.SemaphoreType.DMA,
    ],
)


def _agg64_body(tab_hbm, src_hbm, dst_hbm, zrow_hbm, out_hbm,
                sidx_v, didx_v, rows_v, rows1_v, tab_sh, acc_sh,
                sem, sem0, sem1):
    c = lax.axis_index("c")
    s = lax.axis_index("s")
    t = c * NSUB + s
    base = NCHB * t + jnp.minimum(t, EXTRA)
    pltpu.async_copy(zrow_hbm, acc_sh.at[pl.ds(s * OSL, OSL), :], sem).wait()
    pltpu.sync_copy(tab_hbm.at[pl.ds(s * OSL, OSL), :],
                    tab_sh.at[pl.ds(s * OSL, OSL), :])
    pltpu.async_copy(src_hbm.at[pl.ds(base, NCHB), :],
                     sidx_v.at[pl.ds(0, NCHB), :], sem).wait()
    pltpu.async_copy(dst_hbm.at[pl.ds(base, NCHB), :],
                     didx_v.at[pl.ds(0, NCHB), :], sem).wait()

    @pl.when(t < EXTRA)
    def _():
        pltpu.async_copy(src_hbm.at[pl.ds(base + NCHB, 1), :],
                         sidx_v.at[pl.ds(NCHB, 1), :], sem).wait()
        pltpu.async_copy(dst_hbm.at[pl.ds(base + NCHB, 1), :],
                         didx_v.at[pl.ds(NCHB, 1), :], sem).wait()

    plsc.subcore_barrier()

    def gstart(j, buf, gsem):
        pltpu.async_copy(tab_sh.at[sidx_v.at[j]], buf, gsem)

    def gwait(j, buf, gsem):
        pltpu.make_async_copy(tab_sh.at[sidx_v.at[j]], buf, gsem).wait()

    def scat(j, buf):
        pltpu.sync_copy(buf, acc_sh.at[didx_v.at[j]], add=True)

    gstart(0, rows_v, sem0)

    def body(k, carry):
        j0 = 2 * k
        gwait(j0, rows_v, sem0)
        gstart(j0 + 1, rows1_v, sem1)
        scat(j0, rows_v)
        gwait(j0 + 1, rows1_v, sem1)

        @pl.when(k < NCHB // 2 - 1)
        def _():
            gstart(j0 + 2, rows_v, sem0)

        scat(j0 + 1, rows1_v)
        return carry

    lax.fori_loop(0, NCHB // 2, body, 0)

    @pl.when(t < EXTRA)
    def _():
        pltpu.async_copy(tab_sh.at[sidx_v.at[NCHB]], rows_v, sem0).wait()
        scat(NCHB, rows_v)

    plsc.subcore_barrier()
    pltpu.sync_copy(acc_sh.at[pl.ds(s * OSL, OSL), :],
                    out_hbm.at[c, pl.ds(s * OSL, OSL), :])


_agg64 = pl.kernel(
    _agg64_body,
    mesh=_MESH,
    compiler_params=_SC_PARAMS,
    out_type=jax.ShapeDtypeStruct((NCORES, N, HID), jnp.float32),
    scratch_types=[
        pltpu.VMEM((NCHB + 1, CHUNK), jnp.int32),
        pltpu.VMEM((NCHB + 1, CHUNK), jnp.int32),
        pltpu.VMEM((CHUNK, HID), jnp.float32),
        pltpu.VMEM((CHUNK, HID), jnp.float32),
        pltpu.VMEM_SHARED((N, HID), jnp.float32),
        pltpu.VMEM_SHARED((N, HID), jnp.float32),
        pltpu.SemaphoreType.DMA,
        pltpu.SemaphoreType.DMA,
        pltpu.SemaphoreType.DMA,
    ],
)


AR = TAB2 // 16


def _agg2_body(tab_hbm, src_hbm, dst_hbm, iota_hbm, out_hbm,
               tab_v, src_v, dst_v, iota_v, acc_v, acc_sh, sem):
    c = lax.axis_index("c")
    s = lax.axis_index("s")
    t = c * NSUB + s
    pltpu.async_copy(tab_hbm, tab_v, sem).wait()
    pltpu.async_copy(src_hbm.at[pl.ds(t * EPT, EPT)], src_v, sem).wait()
    pltpu.async_copy(dst_hbm.at[pl.ds(t * EPT, EPT)], dst_v, sem).wait()
    pltpu.async_copy(iota_hbm, iota_v, sem).wait()
    zero16 = jnp.zeros((16,), jnp.float32)

    def zbody(i, carry):
        acc_v[i, :] = zero16
        return carry

    lax.fori_loop(0, AR, zbody, 0)

    @pl.when(s == 0)
    def _():
        pltpu.sync_copy(acc_v, acc_sh)

    col16 = jnp.full((16,), 15, jnp.int32)

    def body(i, carry):
        sv = src_v[pl.ds(i * 16, 16)] * 2
        dv = dst_v[pl.ds(i * 16, 16)] * 2
        c0 = plsc.load_gather(tab_v, [sv])
        c1 = plsc.load_gather(tab_v, [sv + 1])
        plsc.addupdate_scatter(acc_v, [lax.shift_right_logical(dv, 4),
                                       lax.bitwise_and(dv, col16)], c0)
        dv1 = dv + 1
        plsc.addupdate_scatter(acc_v, [lax.shift_right_logical(dv1, 4),
                                       lax.bitwise_and(dv1, col16)], c1)
        return carry

    lax.fori_loop(0, EPT // 16, body, 0)
    plsc.subcore_barrier()
    for k in range(AR // HCH):
        pltpu.sync_copy(acc_v.at[pl.ds(k * HCH, HCH), :],
                        acc_sh.at[iota_v.at[k]], add=True)
    plsc.subcore_barrier()

    @pl.when(s == 0)
    def _():
        pltpu.sync_copy(acc_sh, out_hbm.at[c])


_agg2 = pl.kernel(
    _agg2_body,
    mesh=_MESH,
    compiler_params=_SC_PARAMS,
    out_type=jax.ShapeDtypeStruct((NCORES, AR, 16), jnp.float32),
    scratch_types=[
        pltpu.VMEM((TAB2,), jnp.float32),
        pltpu.VMEM((EPT,), jnp.int32),
        pltpu.VMEM((EPT,), jnp.int32),
        pltpu.VMEM((AR // HCH, HCH), jnp.int32),
        pltpu.VMEM((AR, 16), jnp.float32),
        pltpu.VMEM_SHARED((AR, 16), jnp.float32),
        pltpu.SemaphoreType.DMA,
    ],
)


_RB = 1000


def _tc1_body(x_ref, w1_ref, hist_ref, h1s_ref, dis_ref):
    deg = jnp.sum(hist_ref[...], axis=1, keepdims=True) + 1.0
    dis = lax.rsqrt(deg)
    h = jnp.dot(x_ref[...], w1_ref[...], preferred_element_type=jnp.float32)
    h1s_ref[...] = h * dis
    dis_ref[...] = dis


def _tc1(x, w1, hist_parts):
    return pl.pallas_call(
        _tc1_body,
        grid=(N // _RB,),
        in_specs=[
            pl.BlockSpec((_RB, F_IN), lambda i: (i, 0)),
            pl.BlockSpec((F_IN, HID), lambda i: (0, 0)),
            pl.BlockSpec((_RB, NCORES), lambda i: (i, 0)),
        ],
        out_specs=[
            pl.BlockSpec((_RB, HID), lambda i: (i, 0)),
            pl.BlockSpec((_RB, 1), lambda i: (i, 0)),
        ],
        out_shape=[
            jax.ShapeDtypeStruct((N, HID), jnp.float32),
            jax.ShapeDtypeStruct((N, 1), jnp.float32),
        ],
    )(x, w1, hist_parts)


def _tc2_body(p_ref, h1s_ref, dis_ref, b1_ref, w2_ref, h2s_ref):
    agg = p_ref[0] + p_ref[1] + h1s_ref[...]
    z = jnp.maximum(agg * dis_ref[...] + b1_ref[...], 0.0)
    h2 = jnp.dot(z, w2_ref[...], preferred_element_type=jnp.float32)
    h2s_ref[...] = h2 * dis_ref[...]


def _tc2(parts, h1s, dis, b1, w2):
    return pl.pallas_call(
        _tc2_body,
        grid=(N // _RB,),
        in_specs=[
            pl.BlockSpec((NCORES, _RB, HID), lambda i: (0, i, 0)),
            pl.BlockSpec((_RB, HID), lambda i: (i, 0)),
            pl.BlockSpec((_RB, 1), lambda i: (i, 0)),
            pl.BlockSpec((1, HID), lambda i: (0, 0)),
            pl.BlockSpec((HID, C_OUT), lambda i: (0, 0)),
        ],
        out_specs=pl.BlockSpec((_RB, C_OUT), lambda i: (i, 0)),
        out_shape=jax.ShapeDtypeStruct((N, C_OUT), jnp.float32),
    )(parts, h1s, dis, b1, w2)


_RB3 = 2000


def _tc3_body(p_ref, h_ref, dis_ref, b_ref, o_ref):
    aggsum = jnp.sum(p_ref[...], axis=1, keepdims=True)
    o_ref[...] = dis_ref[...] * (aggsum + h_ref[...]) + b_ref[...]


def _tc3(parts_t, h2s_flat, dis_rep, b2_rep):
    return pl.pallas_call(
        _tc3_body,
        grid=(TAB2 // _RB3,),
        in_specs=[
            pl.BlockSpec((_RB3, NCORES), lambda i: (i, 0)),
            pl.BlockSpec((_RB3, 1), lambda i: (i, 0)),
            pl.BlockSpec((_RB3, 1), lambda i: (i, 0)),
            pl.BlockSpec((_RB3, 1), lambda i: (i, 0)),
        ],
        out_specs=pl.BlockSpec((_RB3, 1), lambda i: (i, 0)),
        out_shape=jax.ShapeDtypeStruct((TAB2, 1), jnp.float32),
    )(parts_t, h2s_flat, dis_rep, b2_rep)


def kernel(x, edge_index, W1, b1, W2, b2):
    src = edge_index[0]
    dst = edge_index[1]

    hiota = jnp.arange(HR, dtype=jnp.int32).reshape(HR // HCH, HCH)
    aiota = jnp.arange(AR, dtype=jnp.int32).reshape(AR // HCH, HCH)

    hist_parts = _hist(dst, hiota)
    h1s, dis = _tc1(x, W1, hist_parts.reshape(NCORES, N).T)
    parts1 = _agg64(h1s,
                    src.reshape(EROWS, CHUNK),
                    dst.reshape(EROWS, CHUNK),
                    jnp.zeros((OSL, HID), jnp.float32))
    h2s = _tc2(parts1, h1s, dis, b1.reshape(1, -1), W2)
    parts2 = _agg2(h2s.reshape(-1), src, dst, aiota)
    out_flat = _tc3(parts2.reshape(NCORES, TAB2).T,
                    h2s.reshape(-1, 1),
                    jnp.repeat(dis, 2, axis=0),
                    jnp.tile(b2, N).reshape(-1, 1))
    return out_flat.reshape(N, C_OUT)


# speedup vs baseline: 1.0297x; 1.0297x over previous
import jax
import jax.numpy as jnp
from jax import lax
from jax.experimental import pallas as pl
from jax.experimental.pallas import tpu as pltpu
from jax.experimental.pallas import tpu_sc as plsc

N = 10000
E = 320000
F_IN = 128
HID = 64
C_OUT = 2

NCORES = 2
NSUB = 16
NW = NCORES * NSUB
EPT = E // NW
CHUNK = 128
EROWS = E // CHUNK
NCHB = EROWS // NW
EXTRA = EROWS % NW
OSL = N // NSUB
TAB2 = 2 * N

_MESH = plsc.VectorSubcoreMesh(core_axis_name="c", subcore_axis_name="s")
_SC_PARAMS = pltpu.CompilerParams(
    needs_layout_passes=False, use_tc_tiling_on_sc=False)


def _hist_body(dst_hbm, out_hbm, idx_v, acc_v, sem):
    c = lax.axis_index("c")
    s = lax.axis_index("s")
    t = c * NSUB + s
    pltpu.async_copy(dst_hbm.at[pl.ds(t * EPT, EPT)], idx_v, sem).wait()
    zero16 = jnp.zeros((16,), jnp.float32)
    one16 = jnp.ones((16,), jnp.float32)

    def zbody(i, carry):
        acc_v[pl.ds(i * 16, 16)] = zero16
        return carry

    lax.fori_loop(0, N // 16, zbody, 0)

    def body(i, carry):
        d = idx_v[pl.ds(i * 16, 16)]
        plsc.addupdate_scatter(acc_v, [d], one16)
        return carry

    lax.fori_loop(0, EPT // 16, body, 0)
    pltpu.sync_copy(acc_v, out_hbm.at[t])


_hist = pl.kernel(
    _hist_body,
    mesh=_MESH,
    compiler_params=_SC_PARAMS,
    out_type=jax.ShapeDtypeStruct((NW, N), jnp.float32),
    scratch_types=[
        pltpu.VMEM((EPT,), jnp.int32),
        pltpu.VMEM((N,), jnp.float32),
        pltpu.SemaphoreType.DMA,
    ],
)


def _agg64_body(tab_hbm, src_hbm, dst_hbm, zrow_hbm, out_hbm,
                sidx_v, didx_v, rows_v, rows1_v, tab_sh, acc_sh,
                sem, sem0, sem1):
    c = lax.axis_index("c")
    s = lax.axis_index("s")
    t = c * NSUB + s
    base = NCHB * t + jnp.minimum(t, EXTRA)
    pltpu.async_copy(zrow_hbm, acc_sh.at[pl.ds(s * OSL, OSL), :], sem).wait()
    pltpu.sync_copy(tab_hbm.at[pl.ds(s * OSL, OSL), :],
                    tab_sh.at[pl.ds(s * OSL, OSL), :])
    pltpu.async_copy(src_hbm.at[pl.ds(base, NCHB), :],
                     sidx_v.at[pl.ds(0, NCHB), :], sem).wait()
    pltpu.async_copy(dst_hbm.at[pl.ds(base, NCHB), :],
                     didx_v.at[pl.ds(0, NCHB), :], sem).wait()

    @pl.when(t < EXTRA)
    def _():
        pltpu.async_copy(src_hbm.at[pl.ds(base + NCHB, 1), :],
                         sidx_v.at[pl.ds(NCHB, 1), :], sem).wait()
        pltpu.async_copy(dst_hbm.at[pl.ds(base + NCHB, 1), :],
                         didx_v.at[pl.ds(NCHB, 1), :], sem).wait()

    plsc.subcore_barrier()

    def gstart(j, buf, gsem):
        pltpu.async_copy(tab_sh.at[sidx_v.at[j]], buf, gsem)

    def gwait(j, buf, gsem):
        pltpu.make_async_copy(tab_sh.at[sidx_v.at[j]], buf, gsem).wait()

    def scat(j, buf):
        pltpu.sync_copy(buf, acc_sh.at[didx_v.at[j]], add=True)

    gstart(0, rows_v, sem0)

    def body(k, carry):
        j0 = 2 * k
        gwait(j0, rows_v, sem0)
        gstart(j0 + 1, rows1_v, sem1)
        scat(j0, rows_v)
        gwait(j0 + 1, rows1_v, sem1)

        @pl.when(k < NCHB // 2 - 1)
        def _():
            gstart(j0 + 2, rows_v, sem0)

        scat(j0 + 1, rows1_v)
        return carry

    lax.fori_loop(0, NCHB // 2, body, 0)

    @pl.when(t < EXTRA)
    def _():
        pltpu.async_copy(tab_sh.at[sidx_v.at[NCHB]], rows_v, sem0).wait()
        scat(NCHB, rows_v)

    plsc.subcore_barrier()
    pltpu.sync_copy(acc_sh.at[pl.ds(s * OSL, OSL), :],
                    out_hbm.at[c, pl.ds(s * OSL, OSL), :])


_agg64 = pl.kernel(
    _agg64_body,
    mesh=_MESH,
    compiler_params=_SC_PARAMS,
    out_type=jax.ShapeDtypeStruct((NCORES, N, HID), jnp.float32),
    scratch_types=[
        pltpu.VMEM((NCHB + 1, CHUNK), jnp.int32),
        pltpu.VMEM((NCHB + 1, CHUNK), jnp.int32),
        pltpu.VMEM((CHUNK, HID), jnp.float32),
        pltpu.VMEM((CHUNK, HID), jnp.float32),
        pltpu.VMEM_SHARED((N, HID), jnp.float32),
        pltpu.VMEM_SHARED((N, HID), jnp.float32),
        pltpu.SemaphoreType.DMA,
        pltpu.SemaphoreType.DMA,
        pltpu.SemaphoreType.DMA,
    ],
)


def _agg2_body(tab_hbm, src_hbm, dst_hbm, out_hbm,
               tab_v, src_v, dst_v, acc_v, sem):
    c = lax.axis_index("c")
    s = lax.axis_index("s")
    t = c * NSUB + s
    pltpu.async_copy(tab_hbm, tab_v, sem).wait()
    pltpu.async_copy(src_hbm.at[pl.ds(t * EPT, EPT)], src_v, sem).wait()
    pltpu.async_copy(dst_hbm.at[pl.ds(t * EPT, EPT)], dst_v, sem).wait()
    zero16 = jnp.zeros((16,), jnp.float32)

    def zbody(i, carry):
        acc_v[pl.ds(i * 16, 16)] = zero16
        return carry

    lax.fori_loop(0, TAB2 // 16, zbody, 0)

    def body(i, carry):
        sv = src_v[pl.ds(i * 16, 16)] * 2
        dv = dst_v[pl.ds(i * 16, 16)] * 2
        c0 = plsc.load_gather(tab_v, [sv])
        c1 = plsc.load_gather(tab_v, [sv + 1])
        plsc.addupdate_scatter(acc_v, [dv], c0)
        plsc.addupdate_scatter(acc_v, [dv + 1], c1)
        return carry

    lax.fori_loop(0, EPT // 16, body, 0)
    pltpu.sync_copy(acc_v, out_hbm.at[t])


_agg2 = pl.kernel(
    _agg2_body,
    mesh=_MESH,
    compiler_params=_SC_PARAMS,
    out_type=jax.ShapeDtypeStruct((NW, TAB2), jnp.float32),
    scratch_types=[
        pltpu.VMEM((TAB2,), jnp.float32),
        pltpu.VMEM((EPT,), jnp.int32),
        pltpu.VMEM((EPT,), jnp.int32),
        pltpu.VMEM((TAB2,), jnp.float32),
        pltpu.SemaphoreType.DMA,
    ],
)


_RB = 1000


def _tc1_body(x_ref, w1_ref, hist_ref, h1s_ref, dis_ref):
    deg = jnp.sum(hist_ref[...], axis=1, keepdims=True) + 1.0
    dis = lax.rsqrt(deg)
    h = jnp.dot(x_ref[...], w1_ref[...], preferred_element_type=jnp.float32)
    h1s_ref[...] = h * dis
    dis_ref[...] = dis


def _tc1(x, w1, hist_parts):
    return pl.pallas_call(
        _tc1_body,
        grid=(N // _RB,),
        in_specs=[
            pl.BlockSpec((_RB, F_IN), lambda i: (i, 0)),
            pl.BlockSpec((F_IN, HID), lambda i: (0, 0)),
            pl.BlockSpec((_RB, NW), lambda i: (i, 0)),
        ],
        out_specs=[
            pl.BlockSpec((_RB, HID), lambda i: (i, 0)),
            pl.BlockSpec((_RB, 1), lambda i: (i, 0)),
        ],
        out_shape=[
            jax.ShapeDtypeStruct((N, HID), jnp.float32),
            jax.ShapeDtypeStruct((N, 1), jnp.float32),
        ],
    )(x, w1, hist_parts)


def _tc2_body(p_ref, h1s_ref, dis_ref, b1_ref, w2_ref, h2s_ref):
    agg = p_ref[0] + p_ref[1] + h1s_ref[...]
    z = jnp.maximum(agg * dis_ref[...] + b1_ref[...], 0.0)
    h2 = jnp.dot(z, w2_ref[...], preferred_element_type=jnp.float32)
    h2s_ref[...] = h2 * dis_ref[...]


def _tc2(parts, h1s, dis, b1, w2):
    return pl.pallas_call(
        _tc2_body,
        grid=(N // _RB,),
        in_specs=[
            pl.BlockSpec((NCORES, _RB, HID), lambda i: (0, i, 0)),
            pl.BlockSpec((_RB, HID), lambda i: (i, 0)),
            pl.BlockSpec((_RB, 1), lambda i: (i, 0)),
            pl.BlockSpec((1, HID), lambda i: (0, 0)),
            pl.BlockSpec((HID, C_OUT), lambda i: (0, 0)),
        ],
        out_specs=pl.BlockSpec((_RB, C_OUT), lambda i: (i, 0)),
        out_shape=jax.ShapeDtypeStruct((N, C_OUT), jnp.float32),
    )(parts, h1s, dis, b1, w2)


_RB3 = 2000


def _tc3_body(p_ref, h_ref, dis_ref, b_ref, o_ref):
    aggsum = jnp.sum(p_ref[...], axis=1, keepdims=True)
    o_ref[...] = dis_ref[...] * (aggsum + h_ref[...]) + b_ref[...]


def _tc3(parts_t, h2s_flat, dis_rep, b2_rep):
    return pl.pallas_call(
        _tc3_body,
        grid=(TAB2 // _RB3,),
        in_specs=[
            pl.BlockSpec((_RB3, NW), lambda i: (i, 0)),
            pl.BlockSpec((_RB3, 1), lambda i: (i, 0)),
            pl.BlockSpec((_RB3, 1), lambda i: (i, 0)),
            pl.BlockSpec((_RB3, 1), lambda i: (i, 0)),
        ],
        out_specs=pl.BlockSpec((_RB3, 1), lambda i: (i, 0)),
        out_shape=jax.ShapeDtypeStruct((TAB2, 1), jnp.float32),
    )(parts_t, h2s_flat, dis_rep, b2_rep)


def kernel(x, edge_index, W1, b1, W2, b2):
    src = edge_index[0]
    dst = edge_index[1]

    hist_parts = _hist(dst)
    h1s, dis = _tc1(x, W1, hist_parts.T)
    parts1 = _agg64(h1s,
                    src.reshape(EROWS, CHUNK),
                    dst.reshape(EROWS, CHUNK),
                    jnp.zeros((OSL, HID), jnp.float32))
    h2s = _tc2(parts1, h1s, dis, b1.reshape(1, -1), W2)
    parts2 = _agg2(h2s.reshape(-1), src, dst)
    out_flat = _tc3(parts2.T,
                    h2s.reshape(-1, 1),
                    jnp.repeat(dis, 2, axis=0),
                    jnp.tile(b2, N).reshape(-1, 1))
    return out_flat.reshape(N, C_OUT)
